# baseline (speedup 1.0000x reference)
import jax
import jax.numpy as jnp
from jax import lax
from jax.experimental import pallas as pl
from jax.experimental.pallas import tpu as pltpu
from jax.experimental.pallas import tpu_sc as plsc

DIM = 32
NC, NS = 2, 16
NW = NC * NS
VEC = 1024
SB = 1
CHUNK = SB * VEC
NBUF = 3


def _body(idx_hbm, table_hbm, out_hbm, idx_v, rows_v, isem, gsem, wsem):
    n_chunks = idx_hbm.shape[0]
    per_w = n_chunks // NW
    wid = lax.axis_index("s") * NC + lax.axis_index("c")

    def start_idx(j):
        b = j % NBUF
        pltpu.async_copy(idx_hbm.at[wid * per_w + j], idx_v.at[b], isem.at[b])

    def fire(j):
        b = j % NBUF
        pltpu.make_async_copy(idx_hbm.at[0], idx_v.at[b], isem.at[b]).wait()
        pltpu.async_copy(table_hbm.at[idx_v.at[b]], rows_v.at[b], gsem.at[b])

    def retire(j):
        b = j % NBUF
        pltpu.make_async_copy(out_hbm.at[0], rows_v.at[b], gsem.at[b]).wait()
        pltpu.async_copy(rows_v.at[b], out_hbm.at[wid * per_w + j], wsem.at[b])

    def drain_write(b):
        pltpu.make_async_copy(rows_v.at[b], out_hbm.at[0], wsem.at[b]).wait()

    for j in range(NBUF):
        start_idx(j)
    for j in range(NBUF - 1):
        fire(j)

    def body(i, carry):
        b = i % NBUF

        @pl.when(i >= NBUF)
        def _():
            drain_write(b)

        fire(i)
        jr = i - (NBUF - 1)
        retire(jr)

        @pl.when(i + 1 < per_w)
        def _():
            start_idx(jr + NBUF)

        return carry

    lax.fori_loop(NBUF - 1, per_w, body, 0)

    for jr in range(per_w - NBUF + 1, per_w):
        retire(jr)
    for b in range(NBUF):
        drain_write(b)


def kernel(token_ids, weight):
    B, S = token_ids.shape
    n_rows = B * S
    assert n_rows % (CHUNK * NW) == 0
    n_chunks = n_rows // CHUNK
    idx = token_ids.reshape(n_chunks, CHUNK)

    grab = pl.kernel(
        _body,
        out_type=jax.ShapeDtypeStruct((n_chunks, CHUNK, DIM), jnp.float32),
        mesh=plsc.VectorSubcoreMesh(
            core_axis_name="c", subcore_axis_name="s",
            num_cores=NC, num_subcores=NS,
        ),
        scratch_types=[
            pltpu.VMEM((NBUF, CHUNK), jnp.int32),
            pltpu.VMEM((NBUF, CHUNK, DIM), jnp.float32),
            pltpu.SemaphoreType.DMA((NBUF,)),
            pltpu.SemaphoreType.DMA((NBUF,)),
            pltpu.SemaphoreType.DMA((NBUF,)),
        ],
        compiler_params=pltpu.CompilerParams(use_tc_tiling_on_sc=False),
    )
    out = grab(idx, weight)
    return out.reshape(B, S, DIM)
